# baseline (device time: 43822 ns/iter reference)
import functools

import jax
import jax.numpy as jnp
from jax import lax
from jax.experimental import pallas as pl
from jax.experimental.pallas import tpu as pltpu

N_DEV = 16
HEADS_PER_SHARD = 8
GQA_GROUP = 4
DH = 128
SCALE = 0.08838834764831843


def kernel(x, Wq, Wo, Wk, Wv):
    i = lax.axis_index("i")
    x2 = x[0]
    kv_cols = (HEADS_PER_SHARD // GQA_GROUP) * DH
    Wk_s = lax.dynamic_slice(Wk, (0, i * kv_cols), (Wk.shape[0], kv_cols))
    Wv_s = lax.dynamic_slice(Wv, (0, i * kv_cols), (Wv.shape[0], kv_cols))

    seq, d_model = x2.shape

    def body(x_ref, wq_ref, wk_ref, wv_ref, wo_ref, out_ref,
             attn_ref, wq_vmem, wo_vmem, tmp_a, tmp_b,
             cp_sems, send_sems, recv_sems):
        wq_cp = pltpu.make_async_copy(wq_ref, wq_vmem, cp_sems.at[0])
        wq_cp.start()
        wo_cp = pltpu.make_async_copy(wo_ref, wo_vmem, cp_sems.at[1])
        wo_cp.start()
        my = lax.axis_index("i")
        r = my % 4
        zr = my // 4
        base = my - r
        plane_partners = [base + (r + o) % 4 for o in (1, 2, 3)]
        z_partners = [r + 4 * ((zr + o) % 4) for o in (1, 2, 3)]
        partners = plane_partners + z_partners

        quarter = seq // 4
        sub = seq // 16
        q_lo = quarter * r
        s_lo = q_lo + sub * zr

        bf16 = jnp.bfloat16
        xb = x_ref[...].astype(bf16)
        k = jnp.dot(xb, wk_ref[...].astype(bf16),
                    preferred_element_type=jnp.float32)
        v = jnp.dot(xb, wv_ref[...].astype(bf16),
                    preferred_element_type=jnp.float32)
        wq_cp.wait()
        q = jnp.dot(xb, wq_vmem[...].astype(bf16),
                    preferred_element_type=jnp.float32)

        kb = k.astype(bf16)
        vb = v.astype(bf16)
        qb = q.astype(bf16)
        outs = []
        for h in range(HEADS_PER_SHARD):
            g = h // GQA_GROUP
            qh = qb[:, h * DH:(h + 1) * DH]
            kh = kb[:, g * DH:(g + 1) * DH]
            vh = vb[:, g * DH:(g + 1) * DH]
            s = jnp.dot(qh, kh.T, preferred_element_type=jnp.float32) * SCALE
            m = jnp.max(s, axis=1, keepdims=True)
            p = jnp.exp(s - m)
            l = jnp.sum(p, axis=1, keepdims=True)
            pv = jnp.dot(p.astype(bf16), vh,
                         preferred_element_type=jnp.float32)
            outs.append(pv / l)
        attn_ref[...] = jnp.concatenate(outs, axis=1)

        barrier_sem = pltpu.get_barrier_semaphore()
        for nbr in partners:
            pl.semaphore_signal(
                barrier_sem, inc=1,
                device_id=(nbr,), device_id_type=pl.DeviceIdType.MESH,
            )
        pl.semaphore_wait(barrier_sem, 6)

        wo_cp.wait()
        wob = wo_vmem[...].astype(bf16)

        def compute_quarter(row_lo):
            out_ref[pl.ds(row_lo, quarter)] = jnp.dot(
                attn_ref[pl.ds(row_lo, quarter)].astype(bf16), wob,
                preferred_element_type=jnp.float32,
            )

        def quad_exchange(sem_base, o, partner, src, dst):
            slot = 4 - o
            rdma = pltpu.make_async_remote_copy(
                src_ref=src,
                dst_ref=dst,
                send_sem=send_sems.at[sem_base + o - 1],
                recv_sem=recv_sems.at[sem_base + slot - 1],
                device_id=(partner,),
                device_id_type=pl.DeviceIdType.MESH,
            )
            rdma.start()
            return rdma

        descs = []
        for o in (1, 2, 3):
            pr = (r + o) % 4
            compute_quarter(quarter * pr)
            descs.append(quad_exchange(
                0, o, base + pr,
                out_ref.at[pl.ds(quarter * pr, quarter)],
                tmp_a.at[4 - o],
            ))
        compute_quarter(q_lo)
        for d in descs:
            d.wait()
        out_ref[pl.ds(q_lo, quarter)] = (
            out_ref[pl.ds(q_lo, quarter)]
            + tmp_a[1] + tmp_a[2] + tmp_a[3]
        )

        descs = []
        for o in (1, 2, 3):
            pzr = (zr + o) % 4
            descs.append(quad_exchange(
                3, o, r + 4 * pzr,
                out_ref.at[pl.ds(q_lo + sub * pzr, sub)],
                tmp_b.at[4 - o],
            ))
        for d in descs:
            d.wait()
        out_ref[pl.ds(s_lo, sub)] = (
            out_ref[pl.ds(s_lo, sub)] + tmp_b[1] + tmp_b[2] + tmp_b[3]
        )

        descs = []
        for o in (1, 2, 3):
            pzr = (zr + o) % 4
            descs.append(quad_exchange(
                6, o, r + 4 * pzr,
                out_ref.at[pl.ds(s_lo, sub)],
                out_ref.at[pl.ds(s_lo, sub)],
            ))
        for d in descs:
            d.wait()

        descs = []
        for o in (1, 2, 3):
            pr = (r + o) % 4
            descs.append(quad_exchange(
                9, o, base + pr,
                out_ref.at[pl.ds(q_lo, quarter)],
                out_ref.at[pl.ds(q_lo, quarter)],
            ))
        for d in descs:
            d.wait()

        @functools.partial(
            pl.run_scoped, second_barrier=pltpu.SemaphoreType.REGULAR
        )
        def _(second_barrier):
            for nbr in partners:
                pl.semaphore_signal(
                    second_barrier, inc=1,
                    device_id=(nbr,), device_id_type=pl.DeviceIdType.MESH,
                )
            pl.semaphore_wait(second_barrier, 6)

    out = pl.pallas_call(
        body,
        out_shape=jax.ShapeDtypeStruct((seq, d_model), jnp.float32),
        in_specs=[
            pl.BlockSpec(memory_space=pltpu.VMEM),
            pl.BlockSpec(memory_space=pltpu.MemorySpace.HBM),
            pl.BlockSpec(memory_space=pltpu.VMEM),
            pl.BlockSpec(memory_space=pltpu.VMEM),
            pl.BlockSpec(memory_space=pltpu.MemorySpace.HBM),
        ],
        out_specs=pl.BlockSpec(memory_space=pltpu.VMEM),
        scratch_shapes=[
            pltpu.VMEM((seq, d_model), jnp.float32),
            pltpu.VMEM((d_model, d_model), jnp.float32),
            pltpu.VMEM((d_model, d_model), jnp.float32),
            pltpu.VMEM((4, seq // 4, d_model), jnp.float32),
            pltpu.VMEM((4, seq // 16, d_model), jnp.float32),
            pltpu.SemaphoreType.DMA((2,)),
            pltpu.SemaphoreType.DMA((12,)),
            pltpu.SemaphoreType.DMA((12,)),
        ],
        compiler_params=pltpu.CompilerParams(collective_id=0),
    )(x2, Wq, Wk_s, Wv_s, Wo)
    return out[None]


# device time: 42483 ns/iter; 1.0315x vs baseline; 1.0315x over previous
import functools

import jax
import jax.numpy as jnp
from jax import lax
from jax.experimental import pallas as pl
from jax.experimental.pallas import tpu as pltpu

N_DEV = 16
HEADS_PER_SHARD = 8
GQA_GROUP = 4
DH = 128
SCALE = 0.08838834764831843


def kernel(x, Wq, Wo, Wk, Wv):
    x2 = x[0]
    kv_cols = (HEADS_PER_SHARD // GQA_GROUP) * DH
    seq, d_model = x2.shape

    def body(x_ref, wq_ref, wk_ref, wv_ref, wo_ref, out_ref,
             attn_ref, wq_vmem, wo_vmem, wk_vmem, wv_vmem, tmp_a, tmp_b,
             cp_sems, send_sems, recv_sems):
        my0 = lax.axis_index("i")
        kv_lo = my0 * kv_cols
        wk_cp = pltpu.make_async_copy(
            wk_ref.at[:, pl.ds(kv_lo, kv_cols)], wk_vmem, cp_sems.at[2])
        wk_cp.start()
        wv_cp = pltpu.make_async_copy(
            wv_ref.at[:, pl.ds(kv_lo, kv_cols)], wv_vmem, cp_sems.at[3])
        wv_cp.start()
        wq_cp = pltpu.make_async_copy(wq_ref, wq_vmem, cp_sems.at[0])
        wq_cp.start()
        wo_cp = pltpu.make_async_copy(wo_ref, wo_vmem, cp_sems.at[1])
        wo_cp.start()
        my = lax.axis_index("i")
        r = my % 4
        zr = my // 4
        base = my - r
        plane_partners = [base + (r + o) % 4 for o in (1, 2, 3)]
        z_partners = [r + 4 * ((zr + o) % 4) for o in (1, 2, 3)]
        partners = plane_partners + z_partners

        quarter = seq // 4
        sub = seq // 16
        q_lo = quarter * r
        s_lo = q_lo + sub * zr

        bf16 = jnp.bfloat16
        xb = x_ref[...].astype(bf16)
        wk_cp.wait()
        k = jnp.dot(xb, wk_vmem[...].astype(bf16),
                    preferred_element_type=jnp.float32)
        wv_cp.wait()
        v = jnp.dot(xb, wv_vmem[...].astype(bf16),
                    preferred_element_type=jnp.float32)
        wq_cp.wait()
        q = jnp.dot(xb, wq_vmem[...].astype(bf16),
                    preferred_element_type=jnp.float32)

        kb = k.astype(bf16)
        vb = v.astype(bf16)
        qb = q.astype(bf16)
        outs = []
        for h in range(HEADS_PER_SHARD):
            g = h // GQA_GROUP
            qh = qb[:, h * DH:(h + 1) * DH]
            kh = kb[:, g * DH:(g + 1) * DH]
            vh = vb[:, g * DH:(g + 1) * DH]
            s = jnp.dot(qh, kh.T, preferred_element_type=jnp.float32) * SCALE
            m = jnp.max(s, axis=1, keepdims=True)
            p = jnp.exp(s - m)
            l = jnp.sum(p, axis=1, keepdims=True)
            pv = jnp.dot(p.astype(bf16), vh,
                         preferred_element_type=jnp.float32)
            outs.append(pv / l)
        attn_ref[...] = jnp.concatenate(outs, axis=1)

        barrier_sem = pltpu.get_barrier_semaphore()
        for nbr in partners:
            pl.semaphore_signal(
                barrier_sem, inc=1,
                device_id=(nbr,), device_id_type=pl.DeviceIdType.MESH,
            )
        pl.semaphore_wait(barrier_sem, 6)

        wo_cp.wait()
        wob = wo_vmem[...].astype(bf16)

        def compute_quarter(row_lo):
            out_ref[pl.ds(row_lo, quarter)] = jnp.dot(
                attn_ref[pl.ds(row_lo, quarter)].astype(bf16), wob,
                preferred_element_type=jnp.float32,
            )

        def quad_exchange(sem_base, o, partner, src, dst):
            slot = 4 - o
            rdma = pltpu.make_async_remote_copy(
                src_ref=src,
                dst_ref=dst,
                send_sem=send_sems.at[sem_base + o - 1],
                recv_sem=recv_sems.at[sem_base + slot - 1],
                device_id=(partner,),
                device_id_type=pl.DeviceIdType.MESH,
            )
            rdma.start()
            return rdma

        descs = []
        for o in (1, 2, 3):
            pr = (r + o) % 4
            compute_quarter(quarter * pr)
            descs.append(quad_exchange(
                0, o, base + pr,
                out_ref.at[pl.ds(quarter * pr, quarter)],
                tmp_a.at[4 - o],
            ))
        compute_quarter(q_lo)
        for d in descs:
            d.wait()
        out_ref[pl.ds(q_lo, quarter)] = (
            out_ref[pl.ds(q_lo, quarter)]
            + tmp_a[1] + tmp_a[2] + tmp_a[3]
        )

        descs = []
        for o in (1, 2, 3):
            pzr = (zr + o) % 4
            descs.append(quad_exchange(
                3, o, r + 4 * pzr,
                out_ref.at[pl.ds(q_lo + sub * pzr, sub)],
                tmp_b.at[4 - o],
            ))
        for d in descs:
            d.wait()
        out_ref[pl.ds(s_lo, sub)] = (
            out_ref[pl.ds(s_lo, sub)] + tmp_b[1] + tmp_b[2] + tmp_b[3]
        )

        descs = []
        for o in (1, 2, 3):
            pzr = (zr + o) % 4
            descs.append(quad_exchange(
                6, o, r + 4 * pzr,
                out_ref.at[pl.ds(s_lo, sub)],
                out_ref.at[pl.ds(s_lo, sub)],
            ))
        for d in descs:
            d.wait()

        descs = []
        for o in (1, 2, 3):
            pr = (r + o) % 4
            descs.append(quad_exchange(
                9, o, base + pr,
                out_ref.at[pl.ds(q_lo, quarter)],
                out_ref.at[pl.ds(q_lo, quarter)],
            ))
        for d in descs:
            d.wait()

        @functools.partial(
            pl.run_scoped, second_barrier=pltpu.SemaphoreType.REGULAR
        )
        def _(second_barrier):
            for nbr in partners:
                pl.semaphore_signal(
                    second_barrier, inc=1,
                    device_id=(nbr,), device_id_type=pl.DeviceIdType.MESH,
                )
            pl.semaphore_wait(second_barrier, 6)

    out = pl.pallas_call(
        body,
        out_shape=jax.ShapeDtypeStruct((seq, d_model), jnp.float32),
        in_specs=[
            pl.BlockSpec(memory_space=pltpu.VMEM),
            pl.BlockSpec(memory_space=pltpu.MemorySpace.HBM),
            pl.BlockSpec(memory_space=pltpu.MemorySpace.HBM),
            pl.BlockSpec(memory_space=pltpu.MemorySpace.HBM),
            pl.BlockSpec(memory_space=pltpu.MemorySpace.HBM),
        ],
        out_specs=pl.BlockSpec(memory_space=pltpu.VMEM),
        scratch_shapes=[
            pltpu.VMEM((seq, d_model), jnp.float32),
            pltpu.VMEM((d_model, d_model), jnp.float32),
            pltpu.VMEM((d_model, d_model), jnp.float32),
            pltpu.VMEM((d_model, kv_cols), jnp.float32),
            pltpu.VMEM((d_model, kv_cols), jnp.float32),
            pltpu.VMEM((4, seq // 4, d_model), jnp.float32),
            pltpu.VMEM((4, seq // 16, d_model), jnp.float32),
            pltpu.SemaphoreType.DMA((4,)),
            pltpu.SemaphoreType.DMA((12,)),
            pltpu.SemaphoreType.DMA((12,)),
        ],
        compiler_params=pltpu.CompilerParams(collective_id=0),
    )(x2, Wq, Wk, Wv, Wo)
    return out[None]


# device time: 34762 ns/iter; 1.2606x vs baseline; 1.2221x over previous
import functools

import jax
import jax.numpy as jnp
from jax import lax
from jax.experimental import pallas as pl
from jax.experimental.pallas import tpu as pltpu

N_DEV = 16
HEADS_PER_SHARD = 8
GQA_GROUP = 4
DH = 128
SCALE = 0.08838834764831843


def kernel(x, Wq, Wo, Wk, Wv):
    x2 = x[0]
    kv_cols = (HEADS_PER_SHARD // GQA_GROUP) * DH
    seq, d_model = x2.shape

    def body(x_ref, wq_ref, wk_ref, wv_ref, wo_ref, out_ref,
             attn_ref, wq_vmem, wo_vmem, wk_vmem, wv_vmem, outb,
             tmp_a, tmp_b, cp_sems, send_sems, recv_sems):
        my0 = lax.axis_index("i")
        kv_lo = my0 * kv_cols
        wk_cp = pltpu.make_async_copy(
            wk_ref.at[:, pl.ds(kv_lo, kv_cols)], wk_vmem, cp_sems.at[2])
        wk_cp.start()
        wv_cp = pltpu.make_async_copy(
            wv_ref.at[:, pl.ds(kv_lo, kv_cols)], wv_vmem, cp_sems.at[3])
        wv_cp.start()
        wq_cp = pltpu.make_async_copy(wq_ref, wq_vmem, cp_sems.at[0])
        wq_cp.start()
        wo_cp = pltpu.make_async_copy(wo_ref, wo_vmem, cp_sems.at[1])
        wo_cp.start()
        my = lax.axis_index("i")
        r = my % 4
        zr = my // 4
        base = my - r
        plane_partners = [base + (r + o) % 4 for o in (1, 2, 3)]
        z_partners = [r + 4 * ((zr + o) % 4) for o in (1, 2, 3)]
        partners = plane_partners + z_partners

        quarter = seq // 4
        sub = seq // 16
        q_lo = quarter * r
        s_lo = q_lo + sub * zr

        bf16 = jnp.bfloat16
        xb = x_ref[...].astype(bf16)
        wk_cp.wait()
        k = jnp.dot(xb, wk_vmem[...].astype(bf16),
                    preferred_element_type=jnp.float32)
        wv_cp.wait()
        v = jnp.dot(xb, wv_vmem[...].astype(bf16),
                    preferred_element_type=jnp.float32)
        wq_cp.wait()
        q = jnp.dot(xb, wq_vmem[...].astype(bf16),
                    preferred_element_type=jnp.float32)

        kb = k.astype(bf16)
        vb = v.astype(bf16)
        qb = q.astype(bf16)
        outs = []
        for h in range(HEADS_PER_SHARD):
            g = h // GQA_GROUP
            qh = qb[:, h * DH:(h + 1) * DH]
            kh = kb[:, g * DH:(g + 1) * DH]
            vh = vb[:, g * DH:(g + 1) * DH]
            s = jnp.dot(qh, kh.T, preferred_element_type=jnp.float32) * SCALE
            m = jnp.max(s, axis=1, keepdims=True)
            p = jnp.exp(s - m)
            l = jnp.sum(p, axis=1, keepdims=True)
            pv = jnp.dot(p.astype(bf16), vh,
                         preferred_element_type=jnp.float32)
            outs.append(pv / l)
        attn_ref[...] = jnp.concatenate(outs, axis=1)

        barrier_sem = pltpu.get_barrier_semaphore()
        for nbr in partners:
            pl.semaphore_signal(
                barrier_sem, inc=1,
                device_id=(nbr,), device_id_type=pl.DeviceIdType.MESH,
            )
        pl.semaphore_wait(barrier_sem, 6)

        wo_cp.wait()
        wob = wo_vmem[...].astype(bf16)

        def compute_quarter_send(row_lo):
            outb[pl.ds(row_lo, quarter)] = jnp.dot(
                attn_ref[pl.ds(row_lo, quarter)].astype(bf16), wob,
                preferred_element_type=jnp.float32,
            ).astype(bf16)

        def compute_quarter_mine(row_lo):
            out_ref[pl.ds(row_lo, quarter)] = jnp.dot(
                attn_ref[pl.ds(row_lo, quarter)].astype(bf16), wob,
                preferred_element_type=jnp.float32,
            )

        def quad_exchange(sem_base, o, partner, src, dst):
            slot = 4 - o
            rdma = pltpu.make_async_remote_copy(
                src_ref=src,
                dst_ref=dst,
                send_sem=send_sems.at[sem_base + o - 1],
                recv_sem=recv_sems.at[sem_base + slot - 1],
                device_id=(partner,),
                device_id_type=pl.DeviceIdType.MESH,
            )
            rdma.start()
            return rdma

        descs = []
        for o in (1, 2, 3):
            pr = (r + o) % 4
            compute_quarter_send(quarter * pr)
            descs.append(quad_exchange(
                0, o, base + pr,
                outb.at[pl.ds(quarter * pr, quarter)],
                tmp_a.at[4 - o],
            ))
        compute_quarter_mine(q_lo)
        for d in descs:
            d.wait()
        out_ref[pl.ds(q_lo, quarter)] = (
            out_ref[pl.ds(q_lo, quarter)]
            + tmp_a[1].astype(jnp.float32)
            + tmp_a[2].astype(jnp.float32)
            + tmp_a[3].astype(jnp.float32)
        )

        descs = []
        for o in (1, 2, 3):
            pzr = (zr + o) % 4
            outb[pl.ds(q_lo + sub * pzr, sub)] = (
                out_ref[pl.ds(q_lo + sub * pzr, sub)].astype(bf16)
            )
            descs.append(quad_exchange(
                3, o, r + 4 * pzr,
                outb.at[pl.ds(q_lo + sub * pzr, sub)],
                tmp_b.at[4 - o],
            ))
        for d in descs:
            d.wait()
        out_ref[pl.ds(s_lo, sub)] = (
            out_ref[pl.ds(s_lo, sub)]
            + tmp_b[1].astype(jnp.float32)
            + tmp_b[2].astype(jnp.float32)
            + tmp_b[3].astype(jnp.float32)
        )

        outb[pl.ds(s_lo, sub)] = out_ref[pl.ds(s_lo, sub)].astype(bf16)
        descs = []
        for o in (1, 2, 3):
            pzr = (zr + o) % 4
            descs.append(quad_exchange(
                6, o, r + 4 * pzr,
                outb.at[pl.ds(s_lo, sub)],
                outb.at[pl.ds(s_lo, sub)],
            ))
        for d in descs:
            d.wait()

        descs = []
        for o in (1, 2, 3):
            pr = (r + o) % 4
            descs.append(quad_exchange(
                9, o, base + pr,
                outb.at[pl.ds(q_lo, quarter)],
                outb.at[pl.ds(q_lo, quarter)],
            ))
        for d in descs:
            d.wait()

        out_ref[...] = outb[...].astype(jnp.float32)

        @functools.partial(
            pl.run_scoped, second_barrier=pltpu.SemaphoreType.REGULAR
        )
        def _(second_barrier):
            for nbr in partners:
                pl.semaphore_signal(
                    second_barrier, inc=1,
                    device_id=(nbr,), device_id_type=pl.DeviceIdType.MESH,
                )
            pl.semaphore_wait(second_barrier, 6)

    out = pl.pallas_call(
        body,
        out_shape=jax.ShapeDtypeStruct((seq, d_model), jnp.float32),
        in_specs=[
            pl.BlockSpec(memory_space=pltpu.VMEM),
            pl.BlockSpec(memory_space=pltpu.MemorySpace.HBM),
            pl.BlockSpec(memory_space=pltpu.MemorySpace.HBM),
            pl.BlockSpec(memory_space=pltpu.MemorySpace.HBM),
            pl.BlockSpec(memory_space=pltpu.MemorySpace.HBM),
        ],
        out_specs=pl.BlockSpec(memory_space=pltpu.VMEM),
        scratch_shapes=[
            pltpu.VMEM((seq, d_model), jnp.float32),
            pltpu.VMEM((d_model, d_model), jnp.float32),
            pltpu.VMEM((d_model, d_model), jnp.float32),
            pltpu.VMEM((d_model, kv_cols), jnp.float32),
            pltpu.VMEM((d_model, kv_cols), jnp.float32),
            pltpu.VMEM((seq, d_model), jnp.bfloat16),
            pltpu.VMEM((4, seq // 4, d_model), jnp.bfloat16),
            pltpu.VMEM((4, seq // 16, d_model), jnp.bfloat16),
            pltpu.SemaphoreType.DMA((4,)),
            pltpu.SemaphoreType.DMA((12,)),
            pltpu.SemaphoreType.DMA((12,)),
        ],
        compiler_params=pltpu.CompilerParams(collective_id=0),
    )(x2, Wq, Wk, Wv, Wo)
    return out[None]


# device time: 34713 ns/iter; 1.2624x vs baseline; 1.0014x over previous
import functools

import jax
import jax.numpy as jnp
from jax import lax
from jax.experimental import pallas as pl
from jax.experimental.pallas import tpu as pltpu

N_DEV = 16
HEADS_PER_SHARD = 8
GQA_GROUP = 4
DH = 128
SCALE = 0.08838834764831843


def kernel(x, Wq, Wo, Wk, Wv):
    x2 = x[0]
    kv_cols = (HEADS_PER_SHARD // GQA_GROUP) * DH
    seq, d_model = x2.shape

    def body(x_ref, wq_ref, wk_ref, wv_ref, wo_ref, out_ref,
             attn_ref, wq_vmem, wo_vmem, wk_vmem, wv_vmem, outb,
             tmp_a, tmp_b, cp_sems, send_sems, recv_sems):
        my0 = lax.axis_index("i")
        kv_lo = my0 * kv_cols
        wk_cp = pltpu.make_async_copy(
            wk_ref.at[:, pl.ds(kv_lo, kv_cols)], wk_vmem, cp_sems.at[2])
        wk_cp.start()
        wv_cp = pltpu.make_async_copy(
            wv_ref.at[:, pl.ds(kv_lo, kv_cols)], wv_vmem, cp_sems.at[3])
        wv_cp.start()
        wq_cp = pltpu.make_async_copy(wq_ref, wq_vmem, cp_sems.at[0])
        wq_cp.start()
        wo_cp = pltpu.make_async_copy(wo_ref, wo_vmem, cp_sems.at[1])
        wo_cp.start()
        my = lax.axis_index("i")
        r = my % 4
        zr = my // 4
        base = my - r
        plane_partners = [base + (r + o) % 4 for o in (1, 2, 3)]
        z_partners = [r + 4 * ((zr + o) % 4) for o in (1, 2, 3)]
        partners = plane_partners + z_partners

        quarter = seq // 4
        sub = seq // 16
        q_lo = quarter * r
        s_lo = q_lo + sub * zr

        bf16 = jnp.bfloat16
        xb = x_ref[...].astype(bf16)
        wk_cp.wait()
        k = jnp.dot(xb, wk_vmem[...].astype(bf16),
                    preferred_element_type=jnp.float32)
        wv_cp.wait()
        v = jnp.dot(xb, wv_vmem[...].astype(bf16),
                    preferred_element_type=jnp.float32)
        wq_cp.wait()
        q = jnp.dot(xb, wq_vmem[...].astype(bf16),
                    preferred_element_type=jnp.float32)

        kb = k.astype(bf16)
        vb = v.astype(bf16)
        qb = q.astype(bf16)
        outs = []
        for h in range(HEADS_PER_SHARD):
            g = h // GQA_GROUP
            qh = qb[:, h * DH:(h + 1) * DH]
            kh = kb[:, g * DH:(g + 1) * DH]
            vh = vb[:, g * DH:(g + 1) * DH]
            s = lax.dot_general(
                qh, kh, (((1,), (1,)), ((), ())),
                preferred_element_type=jnp.float32,
            ) * SCALE
            m = jnp.max(s, axis=1, keepdims=True)
            p = jnp.exp(s - m)
            l = jnp.sum(p, axis=1, keepdims=True)
            pv = jnp.dot(p.astype(bf16), vh,
                         preferred_element_type=jnp.float32)
            outs.append(pv / l)
        attn_ref[...] = jnp.concatenate(outs, axis=1).astype(bf16)

        barrier_sem = pltpu.get_barrier_semaphore()
        for nbr in partners:
            pl.semaphore_signal(
                barrier_sem, inc=1,
                device_id=(nbr,), device_id_type=pl.DeviceIdType.MESH,
            )
        pl.semaphore_wait(barrier_sem, 6)

        wo_cp.wait()
        wob = wo_vmem[...].astype(bf16)

        def compute_quarter_send(row_lo):
            outb[pl.ds(row_lo, quarter)] = jnp.dot(
                attn_ref[pl.ds(row_lo, quarter)], wob,
                preferred_element_type=jnp.float32,
            ).astype(bf16)

        def compute_quarter_mine(row_lo):
            out_ref[pl.ds(row_lo, quarter)] = jnp.dot(
                attn_ref[pl.ds(row_lo, quarter)], wob,
                preferred_element_type=jnp.float32,
            )

        def quad_exchange(sem_base, o, partner, src, dst):
            slot = 4 - o
            rdma = pltpu.make_async_remote_copy(
                src_ref=src,
                dst_ref=dst,
                send_sem=send_sems.at[sem_base + o - 1],
                recv_sem=recv_sems.at[sem_base + slot - 1],
                device_id=(partner,),
                device_id_type=pl.DeviceIdType.MESH,
            )
            rdma.start()
            return rdma

        descs = []
        for o in (1, 2, 3):
            pr = (r + o) % 4
            compute_quarter_send(quarter * pr)
            descs.append(quad_exchange(
                0, o, base + pr,
                outb.at[pl.ds(quarter * pr, quarter)],
                tmp_a.at[4 - o],
            ))
        compute_quarter_mine(q_lo)
        for d in descs:
            d.wait()
        out_ref[pl.ds(q_lo, quarter)] = (
            out_ref[pl.ds(q_lo, quarter)]
            + tmp_a[1].astype(jnp.float32)
            + tmp_a[2].astype(jnp.float32)
            + tmp_a[3].astype(jnp.float32)
        )

        descs = []
        for o in (1, 2, 3):
            pzr = (zr + o) % 4
            outb[pl.ds(q_lo + sub * pzr, sub)] = (
                out_ref[pl.ds(q_lo + sub * pzr, sub)].astype(bf16)
            )
            descs.append(quad_exchange(
                3, o, r + 4 * pzr,
                outb.at[pl.ds(q_lo + sub * pzr, sub)],
                tmp_b.at[4 - o],
            ))
        for d in descs:
            d.wait()
        out_ref[pl.ds(s_lo, sub)] = (
            out_ref[pl.ds(s_lo, sub)]
            + tmp_b[1].astype(jnp.float32)
            + tmp_b[2].astype(jnp.float32)
            + tmp_b[3].astype(jnp.float32)
        )

        outb[pl.ds(s_lo, sub)] = out_ref[pl.ds(s_lo, sub)].astype(bf16)
        descs = []
        for o in (1, 2, 3):
            pzr = (zr + o) % 4
            descs.append(quad_exchange(
                6, o, r + 4 * pzr,
                outb.at[pl.ds(s_lo, sub)],
                outb.at[pl.ds(s_lo, sub)],
            ))
        for d in descs:
            d.wait()

        descs = []
        for o in (1, 2, 3):
            pr = (r + o) % 4
            descs.append(quad_exchange(
                9, o, base + pr,
                outb.at[pl.ds(q_lo, quarter)],
                outb.at[pl.ds(q_lo, quarter)],
            ))
        for d in descs:
            d.wait()

        out_ref[...] = outb[...].astype(jnp.float32)

        @functools.partial(
            pl.run_scoped, second_barrier=pltpu.SemaphoreType.REGULAR
        )
        def _(second_barrier):
            for nbr in partners:
                pl.semaphore_signal(
                    second_barrier, inc=1,
                    device_id=(nbr,), device_id_type=pl.DeviceIdType.MESH,
                )
            pl.semaphore_wait(second_barrier, 6)

    out = pl.pallas_call(
        body,
        out_shape=jax.ShapeDtypeStruct((seq, d_model), jnp.float32),
        in_specs=[
            pl.BlockSpec(memory_space=pltpu.VMEM),
            pl.BlockSpec(memory_space=pltpu.MemorySpace.HBM),
            pl.BlockSpec(memory_space=pltpu.MemorySpace.HBM),
            pl.BlockSpec(memory_space=pltpu.MemorySpace.HBM),
            pl.BlockSpec(memory_space=pltpu.MemorySpace.HBM),
        ],
        out_specs=pl.BlockSpec(memory_space=pltpu.VMEM),
        scratch_shapes=[
            pltpu.VMEM((seq, d_model), jnp.bfloat16),
            pltpu.VMEM((d_model, d_model), jnp.float32),
            pltpu.VMEM((d_model, d_model), jnp.float32),
            pltpu.VMEM((d_model, kv_cols), jnp.float32),
            pltpu.VMEM((d_model, kv_cols), jnp.float32),
            pltpu.VMEM((seq, d_model), jnp.bfloat16),
            pltpu.VMEM((4, seq // 4, d_model), jnp.bfloat16),
            pltpu.VMEM((4, seq // 16, d_model), jnp.bfloat16),
            pltpu.SemaphoreType.DMA((4,)),
            pltpu.SemaphoreType.DMA((12,)),
            pltpu.SemaphoreType.DMA((12,)),
        ],
        compiler_params=pltpu.CompilerParams(collective_id=0),
    )(x2, Wq, Wk, Wv, Wo)
    return out[None]
